# baseline (speedup 1.0000x reference)
import functools

import jax
import jax.numpy as jnp
from jax import lax
from jax.experimental import pallas as pl
from jax.experimental.pallas import tpu as pltpu
from jax.experimental.pallas import tpu_sc as plsc

N_NODES = 10000
NP = 10240
D = 128
NC = 2
NS = 16
NW = NC * NS
CH = 128
EC = 80
LC = 51
STRIPE = NP // NS
DUMMY = N_NODES

_mesh = plsc.VectorSubcoreMesh(core_axis_name="c", subcore_axis_name="s")


@functools.partial(
    pl.kernel,
    out_type=jax.ShapeDtypeStruct((NC, NP), jnp.float32),
    mesh=_mesh,
    scratch_types=[
        pltpu.VMEM((EC, CH), jnp.int32),
        pltpu.VMEM((CH,), jnp.float32),
        pltpu.VMEM((STRIPE,), jnp.float32),
        pltpu.VMEM_SHARED((NP,), jnp.float32),
    ],
)
def _sc_degree(dst_hbm, out_hbm, idx_v, ones_v, zeros_v, deg_sp):
    cid = lax.axis_index("c")
    sid = lax.axis_index("s")
    wid = sid * NC + cid

    for k in range(CH // 16):
        ones_v[pl.ds(k * 16, 16)] = jnp.full((16,), 1.0, jnp.float32)

    def _zb(i, carry):
        zeros_v[pl.ds(i * 16, 16)] = jnp.zeros((16,), jnp.float32)
        return carry

    lax.fori_loop(0, STRIPE // 16, _zb, 0)
    pltpu.sync_copy(zeros_v, deg_sp.at[pl.ds(sid * STRIPE, STRIPE)])
    plsc.subcore_barrier()

    pltpu.sync_copy(dst_hbm.at[wid], idx_v)

    def _body(j, carry):
        pltpu.sync_copy(ones_v, deg_sp.at[idx_v.at[j]], add=True)
        return carry

    lax.fori_loop(0, EC, _body, 0)
    plsc.subcore_barrier()
    pltpu.sync_copy(deg_sp.at[pl.ds(sid * STRIPE, STRIPE)],
                    out_hbm.at[cid, pl.ds(sid * STRIPE, STRIPE)])


def _enc_body(x_ref, w_ref, b_ref, cnt_ref, o_ref):
    h = jnp.dot(x_ref[...], w_ref[...], preferred_element_type=jnp.float32)
    h = h + b_ref[...]
    deg = cnt_ref[0, :] + cnt_ref[1, :] + 1.0
    norm = lax.rsqrt(deg)
    o_ref[...] = h * norm[:, None]


_tc_encode = pl.pallas_call(
    _enc_body,
    grid=(NP // 1024,),
    in_specs=[
        pl.BlockSpec((1024, D), lambda i: (i, 0)),
        pl.BlockSpec((D, D), lambda i: (0, 0)),
        pl.BlockSpec((1, D), lambda i: (0, 0)),
        pl.BlockSpec((NC, 1024), lambda i: (0, i)),
    ],
    out_specs=pl.BlockSpec((1024, D), lambda i: (i, 0)),
    out_shape=jax.ShapeDtypeStruct((NP, D), jnp.float32),
)


SCH = 16


@functools.partial(
    pl.kernel,
    out_type=jax.ShapeDtypeStruct((NC, NP, D), jnp.float32),
    mesh=_mesh,
    scratch_types=[
        pltpu.VMEM((SCH, CH), jnp.int32),
        pltpu.VMEM((SCH, CH), jnp.int32),
        pltpu.VMEM((CH, D), jnp.float32),
        pltpu.VMEM((CH, D), jnp.float32),
        pltpu.VMEM_SHARED((NP, D), jnp.float32),
        pltpu.SemaphoreType.DMA,
        pltpu.SemaphoreType.DMA,
        pltpu.SemaphoreType.DMA,
        pltpu.SemaphoreType.DMA,
    ],
)
def _sc_aggregate(p_hbm, src_hbm, dst_hbm, out_hbm,
                  src_v, dst_v, g0, g1, acc_sp, sg0, sg1, ss0, ss1):
    cid = lax.axis_index("c")
    sid = lax.axis_index("s")
    wid = sid * NC + cid

    def _zb(i, carry):
        g0[i // (D // 16), pl.ds((i % (D // 16)) * 16, 16)] = (
            jnp.zeros((16,), jnp.float32))
        return carry

    lax.fori_loop(0, CH * (D // 16), _zb, 0)
    for t in range(STRIPE // CH):
        pltpu.sync_copy(g0, acc_sp.at[pl.ds(sid * STRIPE + t * CH, CH)])
    plsc.subcore_barrier()

    def _super(sc_i, carry):
        pltpu.sync_copy(src_hbm.at[wid, pl.ds(sc_i * SCH, SCH)], src_v)
        pltpu.sync_copy(dst_hbm.at[wid, pl.ds(sc_i * SCH, SCH)], dst_v)
        pltpu.make_async_copy(p_hbm.at[src_v.at[0]], g0, sg0).start()

        def _body(jj, c2):
            j0 = jj * 2
            j1 = j0 + 1
            pltpu.make_async_copy(p_hbm.at[src_v.at[j0]], g0, sg0).wait()
            pltpu.async_copy(g0, acc_sp.at[dst_v.at[j0]], ss0, add=True)

            @pl.when(jj > 0)
            def _():
                pltpu.make_async_copy(
                    g1, acc_sp.at[dst_v.at[j1 - 2]], ss1).wait()

            pltpu.make_async_copy(p_hbm.at[src_v.at[j1]], g1, sg1).start()
            pltpu.make_async_copy(g0, acc_sp.at[dst_v.at[j0]], ss0).wait()

            @pl.when(jj < SCH // 2 - 1)
            def _():
                pltpu.make_async_copy(
                    p_hbm.at[src_v.at[j0 + 2]], g0, sg0).start()

            pltpu.make_async_copy(p_hbm.at[src_v.at[j1]], g1, sg1).wait()
            pltpu.async_copy(g1, acc_sp.at[dst_v.at[j1]], ss1, add=True)
            return c2

        lax.fori_loop(0, SCH // 2, _body, 0)
        pltpu.make_async_copy(g1, acc_sp.at[dst_v.at[SCH - 1]], ss1).wait()
        return carry

    lax.fori_loop(0, EC // SCH, _super, 0)

    plsc.subcore_barrier()
    pltpu.sync_copy(acc_sp.at[pl.ds(sid * STRIPE, STRIPE)],
                    out_hbm.at[cid, pl.ds(sid * STRIPE, STRIPE)])


def _fin_body(acc_ref, p_ref, cnt_ref, o_ref):
    deg = cnt_ref[0, :] + cnt_ref[1, :] + 1.0
    norm = lax.rsqrt(deg)
    t = (acc_ref[0] + acc_ref[1] + p_ref[...]) * norm[:, None]
    o_ref[...] = jnp.maximum(t, 0.0)


_tc_finalize = pl.pallas_call(
    _fin_body,
    grid=(NP // 1024,),
    in_specs=[
        pl.BlockSpec((NC, 1024, D), lambda i: (0, i, 0)),
        pl.BlockSpec((1024, D), lambda i: (i, 0)),
        pl.BlockSpec((NC, 1024), lambda i: (0, i)),
    ],
    out_specs=pl.BlockSpec((1024, D), lambda i: (i, 0)),
    out_shape=jax.ShapeDtypeStruct((NP, D), jnp.float32),
)


@functools.partial(
    pl.kernel,
    out_type=jax.ShapeDtypeStruct((NW, LC, CH), jnp.float32),
    mesh=_mesh,
    compiler_params=pltpu.CompilerParams(needs_layout_passes=False),
    scratch_types=[
        pltpu.VMEM((LC, CH), jnp.int32),
        pltpu.VMEM((LC, CH), jnp.int32),
        pltpu.VMEM((CH, D), jnp.float32),
        pltpu.VMEM((CH, D), jnp.float32),
        pltpu.VMEM((CH, D), jnp.float32),
        pltpu.VMEM((CH, D), jnp.float32),
        pltpu.VMEM((CH, D), jnp.float32),
        pltpu.VMEM((CH, D), jnp.float32),
        pltpu.VMEM((LC, CH), jnp.float32),
        pltpu.SemaphoreType.DMA,
        pltpu.SemaphoreType.DMA,
        pltpu.SemaphoreType.DMA,
    ],
)
def _sc_decode(z_hbm, s_hbm, d_hbm, out_hbm,
               s_v, d_v, a0, b0, a1, b1, a2, b2, o_v, sm0, sm1, sm2):
    cid = lax.axis_index("c")
    sid = lax.axis_index("s")
    wid = sid * NC + cid
    pltpu.sync_copy(s_hbm.at[wid], s_v)
    pltpu.sync_copy(d_hbm.at[wid], d_v)

    def _start(j, abuf, bbuf, sem):
        pltpu.make_async_copy(z_hbm.at[s_v.at[j]], abuf, sem).start()
        pltpu.make_async_copy(z_hbm.at[d_v.at[j]], bbuf, sem).start()

    def _wait(j, abuf, bbuf, sem):
        pltpu.make_async_copy(z_hbm.at[s_v.at[j]], abuf, sem).wait()
        pltpu.make_async_copy(z_hbm.at[d_v.at[j]], bbuf, sem).wait()

    lane = lax.iota(jnp.int32, 16)

    def _compute(j, abuf, bbuf):
        def _grp(g, carry):
            res = jnp.zeros((16,), jnp.float32)
            for i in range(16):
                r = g * 16 + i
                acc = abuf[r, pl.ds(0, 16)] * bbuf[r, pl.ds(0, 16)]
                for k in range(1, D // 16):
                    acc = acc + (abuf[r, pl.ds(k * 16, 16)]
                                 * bbuf[r, pl.ds(k * 16, 16)])
                res = res + jnp.where(lane == i, jnp.sum(acc), 0.0)
            o_v[j, pl.ds(g * 16, 16)] = res
            return carry

        lax.fori_loop(0, CH // 16, _grp, 0)

    bufs = ((a0, b0, sm0), (a1, b1, sm1), (a2, b2, sm2))
    _start(0, a0, b0, sm0)
    _start(1, a1, b1, sm1)

    def _body(jj, carry):
        for u in range(3):
            j = jj * 3 + u
            au, bu, su = bufs[u]
            _wait(j, au, bu, su)
            an, bn, sn = bufs[(u + 2) % 3]

            @pl.when(j + 2 < LC)
            def _():
                _start(j + 2, an, bn, sn)

            _compute(j, au, bu)
        return carry

    lax.fori_loop(0, LC // 3, _body, 0)
    pltpu.sync_copy(o_v, out_hbm.at[wid])


def kernel(x, edge_index, edge_label_index, W, b):
    ne = edge_index.shape[1]
    nl = edge_label_index.shape[1]
    ep = NW * EC * CH
    lp = NW * LC * CH

    src = jnp.concatenate(
        [edge_index[0], jnp.zeros((ep - ne,), jnp.int32)]).reshape(NW, EC, CH)
    dst = jnp.concatenate(
        [edge_index[1], jnp.full((ep - ne,), DUMMY, jnp.int32)]
    ).reshape(NW, EC, CH)
    s_idx = jnp.concatenate(
        [edge_label_index[0], jnp.zeros((lp - nl,), jnp.int32)]
    ).reshape(NW, LC, CH)
    d_idx = jnp.concatenate(
        [edge_label_index[1], jnp.zeros((lp - nl,), jnp.int32)]
    ).reshape(NW, LC, CH)
    x_pad = jnp.concatenate([x, jnp.zeros((NP - x.shape[0], D), x.dtype)])

    cnt = _sc_degree(dst)
    p = _tc_encode(x_pad, W, b.reshape(1, D), cnt)
    acc = _sc_aggregate(p, src, dst)
    z = _tc_finalize(acc, p, cnt)
    logits = _sc_decode(z, s_idx, d_idx)
    return logits.reshape(-1)[:nl]


# speedup vs baseline: 1.5971x; 1.5971x over previous
import functools

import jax
import jax.numpy as jnp
from jax import lax
from jax.experimental import pallas as pl
from jax.experimental.pallas import tpu as pltpu
from jax.experimental.pallas import tpu_sc as plsc

N_NODES = 10000
NP = 10240
D = 128
NC = 2
NS = 16
NW = NC * NS
CH = 128
EC = 80
LC = 50
DW = D // 2
STRIPE = NP // NS
DUMMY = N_NODES

_mesh = plsc.VectorSubcoreMesh(core_axis_name="c", subcore_axis_name="s")


@functools.partial(
    pl.kernel,
    out_type=jax.ShapeDtypeStruct((NC, NP), jnp.float32),
    mesh=_mesh,
    scratch_types=[
        pltpu.VMEM((EC, CH), jnp.int32),
        pltpu.VMEM((CH,), jnp.float32),
        pltpu.VMEM((STRIPE,), jnp.float32),
        pltpu.VMEM_SHARED((NP,), jnp.float32),
    ],
)
def _sc_degree(dst_hbm, out_hbm, idx_v, ones_v, zeros_v, deg_sp):
    cid = lax.axis_index("c")
    sid = lax.axis_index("s")
    wid = sid * NC + cid

    for k in range(CH // 16):
        ones_v[pl.ds(k * 16, 16)] = jnp.full((16,), 1.0, jnp.float32)

    def _zb(i, carry):
        zeros_v[pl.ds(i * 16, 16)] = jnp.zeros((16,), jnp.float32)
        return carry

    lax.fori_loop(0, STRIPE // 16, _zb, 0)
    pltpu.sync_copy(zeros_v, deg_sp.at[pl.ds(sid * STRIPE, STRIPE)])
    plsc.subcore_barrier()

    pltpu.sync_copy(dst_hbm.at[wid], idx_v)

    def _body(j, carry):
        pltpu.sync_copy(ones_v, deg_sp.at[idx_v.at[j]], add=True)
        return carry

    lax.fori_loop(0, EC, _body, 0)
    plsc.subcore_barrier()
    pltpu.sync_copy(deg_sp.at[pl.ds(sid * STRIPE, STRIPE)],
                    out_hbm.at[cid, pl.ds(sid * STRIPE, STRIPE)])


def _enc_body(x_ref, w_ref, b_ref, cnt_ref, o_ref):
    h = jnp.dot(x_ref[...], w_ref[...], preferred_element_type=jnp.float32)
    h = h + b_ref[...]
    deg = cnt_ref[0, :] + cnt_ref[1, :] + 1.0
    norm = lax.rsqrt(deg)
    o_ref[...] = h * norm[:, None]


_tc_encode = pl.pallas_call(
    _enc_body,
    grid=(NP // 1024,),
    in_specs=[
        pl.BlockSpec((1024, D), lambda i: (i, 0)),
        pl.BlockSpec((D, D), lambda i: (0, 0)),
        pl.BlockSpec((1, D), lambda i: (0, 0)),
        pl.BlockSpec((NC, 1024), lambda i: (0, i)),
    ],
    out_specs=pl.BlockSpec((1024, D), lambda i: (i, 0)),
    out_shape=jax.ShapeDtypeStruct((NP, D), jnp.float32),
)


SCH = 16


@functools.partial(
    pl.kernel,
    out_type=jax.ShapeDtypeStruct((NC, NP, D), jnp.float32),
    mesh=_mesh,
    scratch_types=[
        pltpu.VMEM((SCH, CH), jnp.int32),
        pltpu.VMEM((SCH, CH), jnp.int32),
        pltpu.VMEM((CH, D), jnp.float32),
        pltpu.VMEM((CH, D), jnp.float32),
        pltpu.VMEM_SHARED((NP, D), jnp.float32),
        pltpu.SemaphoreType.DMA,
        pltpu.SemaphoreType.DMA,
        pltpu.SemaphoreType.DMA,
        pltpu.SemaphoreType.DMA,
    ],
)
def _sc_aggregate(p_hbm, src_hbm, dst_hbm, out_hbm,
                  src_v, dst_v, g0, g1, acc_sp, sg0, sg1, ss0, ss1):
    cid = lax.axis_index("c")
    sid = lax.axis_index("s")
    wid = sid * NC + cid

    def _zb(i, carry):
        g0[i // (D // 16), pl.ds((i % (D // 16)) * 16, 16)] = (
            jnp.zeros((16,), jnp.float32))
        return carry

    lax.fori_loop(0, CH * (D // 16), _zb, 0)
    for t in range(STRIPE // CH):
        pltpu.sync_copy(g0, acc_sp.at[pl.ds(sid * STRIPE + t * CH, CH)])
    plsc.subcore_barrier()

    def _super(sc_i, carry):
        pltpu.sync_copy(src_hbm.at[wid, pl.ds(sc_i * SCH, SCH)], src_v)
        pltpu.sync_copy(dst_hbm.at[wid, pl.ds(sc_i * SCH, SCH)], dst_v)
        pltpu.make_async_copy(p_hbm.at[src_v.at[0]], g0, sg0).start()

        def _body(jj, c2):
            j0 = jj * 2
            j1 = j0 + 1
            pltpu.make_async_copy(p_hbm.at[src_v.at[j0]], g0, sg0).wait()
            pltpu.async_copy(g0, acc_sp.at[dst_v.at[j0]], ss0, add=True)

            @pl.when(jj > 0)
            def _():
                pltpu.make_async_copy(
                    g1, acc_sp.at[dst_v.at[j1 - 2]], ss1).wait()

            pltpu.make_async_copy(p_hbm.at[src_v.at[j1]], g1, sg1).start()
            pltpu.make_async_copy(g0, acc_sp.at[dst_v.at[j0]], ss0).wait()

            @pl.when(jj < SCH // 2 - 1)
            def _():
                pltpu.make_async_copy(
                    p_hbm.at[src_v.at[j0 + 2]], g0, sg0).start()

            pltpu.make_async_copy(p_hbm.at[src_v.at[j1]], g1, sg1).wait()
            pltpu.async_copy(g1, acc_sp.at[dst_v.at[j1]], ss1, add=True)
            return c2

        lax.fori_loop(0, SCH // 2, _body, 0)
        pltpu.make_async_copy(g1, acc_sp.at[dst_v.at[SCH - 1]], ss1).wait()
        return carry

    lax.fori_loop(0, EC // SCH, _super, 0)

    plsc.subcore_barrier()
    pltpu.sync_copy(acc_sp.at[pl.ds(sid * STRIPE, STRIPE)],
                    out_hbm.at[cid, pl.ds(sid * STRIPE, STRIPE)])


def _fin_body(acc_ref, p_ref, cnt_ref, o_ref):
    deg = cnt_ref[0, :] + cnt_ref[1, :] + 1.0
    norm = lax.rsqrt(deg)
    t = (acc_ref[0] + acc_ref[1] + p_ref[...]) * norm[:, None]
    o_ref[...] = jnp.maximum(t, 0.0).astype(jnp.bfloat16)


_tc_finalize = pl.pallas_call(
    _fin_body,
    grid=(NP // 1024,),
    in_specs=[
        pl.BlockSpec((NC, 1024, D), lambda i: (0, i, 0)),
        pl.BlockSpec((1024, D), lambda i: (i, 0)),
        pl.BlockSpec((NC, 1024), lambda i: (0, i)),
    ],
    out_specs=pl.BlockSpec((1024, D), lambda i: (i, 0)),
    out_shape=jax.ShapeDtypeStruct((NP, D), jnp.bfloat16),
)


@functools.partial(
    pl.kernel,
    out_type=jax.ShapeDtypeStruct((NW, LC, CH), jnp.float32),
    mesh=_mesh,
    compiler_params=pltpu.CompilerParams(
        needs_layout_passes=False, use_tc_tiling_on_sc=False),
    scratch_types=[
        pltpu.VMEM((LC, CH), jnp.int32),
        pltpu.VMEM((LC, CH), jnp.int32),
        pltpu.VMEM((CH, DW), jnp.int32),
        pltpu.VMEM((CH, DW), jnp.int32),
        pltpu.VMEM((CH, DW), jnp.int32),
        pltpu.VMEM((CH, DW), jnp.int32),
        pltpu.VMEM((LC, CH), jnp.float32),
        pltpu.SemaphoreType.DMA,
        pltpu.SemaphoreType.DMA,
    ],
)
def _sc_decode(z_hbm, s_hbm, d_hbm, out_hbm,
               s_v, d_v, a0, b0, a1, b1, o_v, sm0, sm1):
    cid = lax.axis_index("c")
    sid = lax.axis_index("s")
    wid = sid * NC + cid
    pltpu.sync_copy(s_hbm.at[wid], s_v)
    pltpu.sync_copy(d_hbm.at[wid], d_v)

    def _start(j, abuf, bbuf, sem):
        pltpu.make_async_copy(z_hbm.at[s_v.at[j]], abuf, sem).start()
        pltpu.make_async_copy(z_hbm.at[d_v.at[j]], bbuf, sem).start()

    def _wait(j, abuf, bbuf, sem):
        pltpu.make_async_copy(z_hbm.at[s_v.at[j]], abuf, sem).wait()
        pltpu.make_async_copy(z_hbm.at[d_v.at[j]], bbuf, sem).wait()

    lane = lax.iota(jnp.int32, 16)

    def _dotpair(abuf, bbuf, r, k):
        wa = plsc.bitcast(abuf[r, pl.ds(k * 16, 16)], jnp.bfloat16)
        wb = plsc.bitcast(bbuf[r, pl.ds(k * 16, 16)], jnp.bfloat16)
        a_lo, a_hi = plsc.unpack(wa, format=plsc.PackFormat.INTERLEAVED)
        b_lo, b_hi = plsc.unpack(wb, format=plsc.PackFormat.INTERLEAVED)
        return a_lo * b_lo + a_hi * b_hi

    def _compute(j, abuf, bbuf):
        def _grp(g, carry):
            res = jnp.zeros((16,), jnp.float32)
            for i in range(16):
                r = g * 16 + i
                acc = _dotpair(abuf, bbuf, r, 0)
                for k in range(1, DW // 16):
                    acc = acc + _dotpair(abuf, bbuf, r, k)
                res = res + jnp.where(lane == i, jnp.sum(acc), 0.0)
            o_v[j, pl.ds(g * 16, 16)] = res
            return carry

        lax.fori_loop(0, CH // 16, _grp, 0)

    _start(0, a0, b0, sm0)

    def _body(jj, carry):
        j0 = jj * 2
        j1 = j0 + 1
        _wait(j0, a0, b0, sm0)
        _start(j1, a1, b1, sm1)
        _compute(j0, a0, b0)
        _wait(j1, a1, b1, sm1)

        @pl.when(jj < LC // 2 - 1)
        def _():
            _start(j0 + 2, a0, b0, sm0)

        _compute(j1, a1, b1)
        return carry

    lax.fori_loop(0, LC // 2, _body, 0)
    pltpu.sync_copy(o_v, out_hbm.at[wid])


def kernel(x, edge_index, edge_label_index, W, b):
    ne = edge_index.shape[1]
    nl = edge_label_index.shape[1]
    ep = NW * EC * CH
    lp = NW * LC * CH

    src = jnp.concatenate(
        [edge_index[0], jnp.zeros((ep - ne,), jnp.int32)]).reshape(NW, EC, CH)
    dst = jnp.concatenate(
        [edge_index[1], jnp.full((ep - ne,), DUMMY, jnp.int32)]
    ).reshape(NW, EC, CH)
    s_idx = jnp.concatenate(
        [edge_label_index[0], jnp.zeros((lp - nl,), jnp.int32)]
    ).reshape(NW, LC, CH)
    d_idx = jnp.concatenate(
        [edge_label_index[1], jnp.zeros((lp - nl,), jnp.int32)]
    ).reshape(NW, LC, CH)
    x_pad = jnp.concatenate([x, jnp.zeros((NP - x.shape[0], D), x.dtype)])

    cnt = _sc_degree(dst)
    p = _tc_encode(x_pad, W, b.reshape(1, D), cnt)
    acc = _sc_aggregate(p, src, dst)
    z = _tc_finalize(acc, p, cnt)
    z32 = jax.lax.bitcast_convert_type(z.reshape(NP, DW, 2), jnp.int32)
    logits = _sc_decode(z32, s_idx, d_idx)
    return logits.reshape(-1)[:nl]


# speedup vs baseline: 1.8497x; 1.1582x over previous
import functools

import jax
import jax.numpy as jnp
from jax import lax
from jax.experimental import pallas as pl
from jax.experimental.pallas import tpu as pltpu
from jax.experimental.pallas import tpu_sc as plsc

N_NODES = 10000
NP = 10240
D = 128
NC = 2
NS = 16
NW = NC * NS
CH = 128
EC = 80
LC = 50
DW = D // 2
STRIPE = NP // NS
DUMMY = N_NODES

_mesh = plsc.VectorSubcoreMesh(core_axis_name="c", subcore_axis_name="s")


@functools.partial(
    pl.kernel,
    out_type=jax.ShapeDtypeStruct((NC, NP), jnp.float32),
    mesh=_mesh,
    scratch_types=[
        pltpu.VMEM((EC, CH), jnp.int32),
        pltpu.VMEM((CH,), jnp.float32),
        pltpu.VMEM((STRIPE,), jnp.float32),
        pltpu.VMEM_SHARED((NP,), jnp.float32),
    ],
)
def _sc_degree(dst_hbm, out_hbm, idx_v, ones_v, zeros_v, deg_sp):
    cid = lax.axis_index("c")
    sid = lax.axis_index("s")
    wid = sid * NC + cid

    for k in range(CH // 16):
        ones_v[pl.ds(k * 16, 16)] = jnp.full((16,), 1.0, jnp.float32)

    def _zb(i, carry):
        zeros_v[pl.ds(i * 16, 16)] = jnp.zeros((16,), jnp.float32)
        return carry

    lax.fori_loop(0, STRIPE // 16, _zb, 0)
    pltpu.sync_copy(zeros_v, deg_sp.at[pl.ds(sid * STRIPE, STRIPE)])
    plsc.subcore_barrier()

    pltpu.sync_copy(dst_hbm.at[wid], idx_v)

    def _body(j, carry):
        pltpu.sync_copy(ones_v, deg_sp.at[idx_v.at[j]], add=True)
        return carry

    lax.fori_loop(0, EC, _body, 0)
    plsc.subcore_barrier()
    pltpu.sync_copy(deg_sp.at[pl.ds(sid * STRIPE, STRIPE)],
                    out_hbm.at[cid, pl.ds(sid * STRIPE, STRIPE)])


def _enc_body(x_ref, w_ref, b_ref, cnt_ref, o_ref):
    h = jnp.dot(x_ref[...], w_ref[...], preferred_element_type=jnp.float32)
    h = h + b_ref[...]
    deg = cnt_ref[0, :] + cnt_ref[1, :] + 1.0
    norm = lax.rsqrt(deg)
    o_ref[...] = (h * norm[:, None]).astype(jnp.bfloat16)


_tc_encode = pl.pallas_call(
    _enc_body,
    grid=(NP // 1024,),
    in_specs=[
        pl.BlockSpec((1024, D), lambda i: (i, 0)),
        pl.BlockSpec((D, D), lambda i: (0, 0)),
        pl.BlockSpec((1, D), lambda i: (0, 0)),
        pl.BlockSpec((NC, 1024), lambda i: (0, i)),
    ],
    out_specs=pl.BlockSpec((1024, D), lambda i: (i, 0)),
    out_shape=jax.ShapeDtypeStruct((NP, D), jnp.bfloat16),
)


SCH = 16


@functools.partial(
    pl.kernel,
    out_type=jax.ShapeDtypeStruct((NC, NP, D), jnp.float32),
    mesh=_mesh,
    compiler_params=pltpu.CompilerParams(
        needs_layout_passes=False, use_tc_tiling_on_sc=False),
    scratch_types=[
        pltpu.VMEM((SCH, CH), jnp.int32),
        pltpu.VMEM((SCH, CH), jnp.int32),
        pltpu.VMEM((CH, DW), jnp.int32),
        pltpu.VMEM((CH, DW), jnp.int32),
        pltpu.VMEM((CH, D), jnp.float32),
        pltpu.VMEM_SHARED((NP, D), jnp.float32),
        pltpu.SemaphoreType.DMA,
        pltpu.SemaphoreType.DMA,
    ],
)
def _sc_aggregate(p_hbm, src_hbm, dst_hbm, out_hbm,
                  src_v, dst_v, g0, g1, gf, acc_sp, sg0, sg1):
    cid = lax.axis_index("c")
    sid = lax.axis_index("s")
    wid = sid * NC + cid

    def _zb(i, carry):
        gf[i // (D // 16), pl.ds((i % (D // 16)) * 16, 16)] = (
            jnp.zeros((16,), jnp.float32))
        return carry

    lax.fori_loop(0, CH * (D // 16), _zb, 0)
    for t in range(STRIPE // CH):
        pltpu.sync_copy(gf, acc_sp.at[pl.ds(sid * STRIPE + t * CH, CH)])
    plsc.subcore_barrier()

    def _convert(gbuf):
        def _rows(r0, carry):
            for i in range(16):
                r = r0 * 16 + i
                for k in range(DW // 16):
                    w = plsc.bitcast(gbuf[r, pl.ds(k * 16, 16)],
                                     jnp.bfloat16)
                    lo, hi = plsc.unpack(
                        w, format=plsc.PackFormat.INTERLEAVED)
                    gf[r, pl.ds(k * 16, 16)] = lo
                    gf[r, pl.ds(D // 2 + k * 16, 16)] = hi
            return carry

        lax.fori_loop(0, CH // 16, _rows, 0)

    def _super(sc_i, carry):
        pltpu.sync_copy(src_hbm.at[wid, pl.ds(sc_i * SCH, SCH)], src_v)
        pltpu.sync_copy(dst_hbm.at[wid, pl.ds(sc_i * SCH, SCH)], dst_v)
        pltpu.make_async_copy(p_hbm.at[src_v.at[0]], g0, sg0).start()
        pltpu.make_async_copy(p_hbm.at[src_v.at[1]], g1, sg1).start()

        def _body(jj, c2):
            j0 = jj * 2
            j1 = j0 + 1
            pltpu.make_async_copy(p_hbm.at[src_v.at[j0]], g0, sg0).wait()
            _convert(g0)

            @pl.when(jj < SCH // 2 - 1)
            def _():
                pltpu.make_async_copy(
                    p_hbm.at[src_v.at[j0 + 2]], g0, sg0).start()

            pltpu.sync_copy(gf, acc_sp.at[dst_v.at[j0]], add=True)
            pltpu.make_async_copy(p_hbm.at[src_v.at[j1]], g1, sg1).wait()
            _convert(g1)

            @pl.when(jj < SCH // 2 - 1)
            def _():
                pltpu.make_async_copy(
                    p_hbm.at[src_v.at[j1 + 2]], g1, sg1).start()

            pltpu.sync_copy(gf, acc_sp.at[dst_v.at[j1]], add=True)
            return c2

        lax.fori_loop(0, SCH // 2, _body, 0)
        return carry

    lax.fori_loop(0, EC // SCH, _super, 0)

    plsc.subcore_barrier()
    pltpu.sync_copy(acc_sp.at[pl.ds(sid * STRIPE, STRIPE)],
                    out_hbm.at[cid, pl.ds(sid * STRIPE, STRIPE)])


def _fin_body(acc_ref, p_ref, cnt_ref, o_ref):
    deg = cnt_ref[0, :] + cnt_ref[1, :] + 1.0
    norm = lax.rsqrt(deg)
    w = p_ref[...]
    lo = lax.bitcast_convert_type(w << 16, jnp.float32)
    hi = lax.bitcast_convert_type(
        w & jnp.int32(-65536), jnp.float32)
    p_perm = jnp.concatenate([lo, hi], axis=1)
    t = (acc_ref[0] + acc_ref[1] + p_perm) * norm[:, None]
    o_ref[...] = jnp.maximum(t, 0.0).astype(jnp.bfloat16)


_tc_finalize = pl.pallas_call(
    _fin_body,
    grid=(NP // 1024,),
    in_specs=[
        pl.BlockSpec((NC, 1024, D), lambda i: (0, i, 0)),
        pl.BlockSpec((1024, DW), lambda i: (i, 0)),
        pl.BlockSpec((NC, 1024), lambda i: (0, i)),
    ],
    out_specs=pl.BlockSpec((1024, D), lambda i: (i, 0)),
    out_shape=jax.ShapeDtypeStruct((NP, D), jnp.bfloat16),
)


@functools.partial(
    pl.kernel,
    out_type=jax.ShapeDtypeStruct((NW, LC, CH), jnp.float32),
    mesh=_mesh,
    compiler_params=pltpu.CompilerParams(
        needs_layout_passes=False, use_tc_tiling_on_sc=False),
    scratch_types=[
        pltpu.VMEM((LC, CH), jnp.int32),
        pltpu.VMEM((LC, CH), jnp.int32),
        pltpu.VMEM((CH, DW), jnp.int32),
        pltpu.VMEM((CH, DW), jnp.int32),
        pltpu.VMEM((CH, DW), jnp.int32),
        pltpu.VMEM((CH, DW), jnp.int32),
        pltpu.VMEM((LC, CH), jnp.float32),
        pltpu.SemaphoreType.DMA,
        pltpu.SemaphoreType.DMA,
    ],
)
def _sc_decode(z_hbm, s_hbm, d_hbm, out_hbm,
               s_v, d_v, a0, b0, a1, b1, o_v, sm0, sm1):
    cid = lax.axis_index("c")
    sid = lax.axis_index("s")
    wid = sid * NC + cid
    pltpu.sync_copy(s_hbm.at[wid], s_v)
    pltpu.sync_copy(d_hbm.at[wid], d_v)

    def _start(j, abuf, bbuf, sem):
        pltpu.make_async_copy(z_hbm.at[s_v.at[j]], abuf, sem).start()
        pltpu.make_async_copy(z_hbm.at[d_v.at[j]], bbuf, sem).start()

    def _wait(j, abuf, bbuf, sem):
        pltpu.make_async_copy(z_hbm.at[s_v.at[j]], abuf, sem).wait()
        pltpu.make_async_copy(z_hbm.at[d_v.at[j]], bbuf, sem).wait()

    lane = lax.iota(jnp.int32, 16)

    def _dotpair(abuf, bbuf, r, k):
        wa = plsc.bitcast(abuf[r, pl.ds(k * 16, 16)], jnp.bfloat16)
        wb = plsc.bitcast(bbuf[r, pl.ds(k * 16, 16)], jnp.bfloat16)
        a_lo, a_hi = plsc.unpack(wa, format=plsc.PackFormat.INTERLEAVED)
        b_lo, b_hi = plsc.unpack(wb, format=plsc.PackFormat.INTERLEAVED)
        return a_lo * b_lo + a_hi * b_hi

    def _compute(j, abuf, bbuf):
        def _grp(g, carry):
            res = jnp.zeros((16,), jnp.float32)
            for i in range(16):
                r = g * 16 + i
                acc = _dotpair(abuf, bbuf, r, 0)
                for k in range(1, DW // 16):
                    acc = acc + _dotpair(abuf, bbuf, r, k)
                res = res + jnp.where(lane == i, jnp.sum(acc), 0.0)
            o_v[j, pl.ds(g * 16, 16)] = res
            return carry

        lax.fori_loop(0, CH // 16, _grp, 0)

    _start(0, a0, b0, sm0)

    def _body(jj, carry):
        j0 = jj * 2
        j1 = j0 + 1
        _wait(j0, a0, b0, sm0)
        _start(j1, a1, b1, sm1)
        _compute(j0, a0, b0)
        _wait(j1, a1, b1, sm1)

        @pl.when(jj < LC // 2 - 1)
        def _():
            _start(j0 + 2, a0, b0, sm0)

        _compute(j1, a1, b1)
        return carry

    lax.fori_loop(0, LC // 2, _body, 0)
    pltpu.sync_copy(o_v, out_hbm.at[wid])


def kernel(x, edge_index, edge_label_index, W, b):
    ne = edge_index.shape[1]
    nl = edge_label_index.shape[1]
    ep = NW * EC * CH
    lp = NW * LC * CH

    src = jnp.concatenate(
        [edge_index[0], jnp.zeros((ep - ne,), jnp.int32)]).reshape(NW, EC, CH)
    dst = jnp.concatenate(
        [edge_index[1], jnp.full((ep - ne,), DUMMY, jnp.int32)]
    ).reshape(NW, EC, CH)
    s_idx = jnp.concatenate(
        [edge_label_index[0], jnp.zeros((lp - nl,), jnp.int32)]
    ).reshape(NW, LC, CH)
    d_idx = jnp.concatenate(
        [edge_label_index[1], jnp.zeros((lp - nl,), jnp.int32)]
    ).reshape(NW, LC, CH)
    x_pad = jnp.concatenate([x, jnp.zeros((NP - x.shape[0], D), x.dtype)])

    cnt = _sc_degree(dst)
    p = _tc_encode(x_pad, W, b.reshape(1, D), cnt)
    p32 = jax.lax.bitcast_convert_type(p.reshape(NP, DW, 2), jnp.int32)
    acc = _sc_aggregate(p32, src, dst)
    z = _tc_finalize(acc, p32, cnt)
    z32 = jax.lax.bitcast_convert_type(z.reshape(NP, DW, 2), jnp.int32)
    logits = _sc_decode(z32, s_idx, d_idx)
    return logits.reshape(-1)[:nl]


# speedup vs baseline: 2.7530x; 1.4884x over previous
import functools

import jax
import jax.numpy as jnp
from jax import lax
from jax.experimental import pallas as pl
from jax.experimental.pallas import tpu as pltpu
from jax.experimental.pallas import tpu_sc as plsc

N_NODES = 10000
NP = 10240
D = 128
NC = 2
NS = 16
NW = NC * NS
CH = 128
EC = 80
LC = 50
DW = D // 2
STRIPE = NP // NS
DUMMY = N_NODES

_mesh = plsc.VectorSubcoreMesh(core_axis_name="c", subcore_axis_name="s")


@functools.partial(
    pl.kernel,
    out_type=jax.ShapeDtypeStruct((NC, NP), jnp.float32),
    mesh=_mesh,
    scratch_types=[
        pltpu.VMEM((EC, CH), jnp.int32),
        pltpu.VMEM((CH,), jnp.float32),
        pltpu.VMEM((STRIPE,), jnp.float32),
        pltpu.VMEM_SHARED((NP,), jnp.float32),
    ],
)
def _sc_degree(dst_hbm, out_hbm, idx_v, ones_v, zeros_v, deg_sp):
    cid = lax.axis_index("c")
    sid = lax.axis_index("s")
    wid = sid * NC + cid

    for k in range(CH // 16):
        ones_v[pl.ds(k * 16, 16)] = jnp.full((16,), 1.0, jnp.float32)

    def _zb(i, carry):
        zeros_v[pl.ds(i * 16, 16)] = jnp.zeros((16,), jnp.float32)
        return carry

    lax.fori_loop(0, STRIPE // 16, _zb, 0)
    pltpu.sync_copy(zeros_v, deg_sp.at[pl.ds(sid * STRIPE, STRIPE)])
    plsc.subcore_barrier()

    pltpu.sync_copy(dst_hbm.at[wid], idx_v)

    def _body(j, carry):
        pltpu.sync_copy(ones_v, deg_sp.at[idx_v.at[j]], add=True)
        return carry

    lax.fori_loop(0, EC, _body, 0)
    plsc.subcore_barrier()
    pltpu.sync_copy(deg_sp.at[pl.ds(sid * STRIPE, STRIPE)],
                    out_hbm.at[cid, pl.ds(sid * STRIPE, STRIPE)])


def _enc_body(x_ref, w_ref, b_ref, cnt_ref, o_ref):
    h = jnp.dot(x_ref[...], w_ref[...], preferred_element_type=jnp.float32)
    h = h + b_ref[...]
    deg = cnt_ref[0, :] + cnt_ref[1, :] + 1.0
    norm = lax.rsqrt(deg)
    o_ref[...] = (h * norm[:, None]).astype(jnp.bfloat16)


_tc_encode = pl.pallas_call(
    _enc_body,
    grid=(NP // 1024,),
    in_specs=[
        pl.BlockSpec((1024, D), lambda i: (i, 0)),
        pl.BlockSpec((D, D), lambda i: (0, 0)),
        pl.BlockSpec((1, D), lambda i: (0, 0)),
        pl.BlockSpec((NC, 1024), lambda i: (0, i)),
    ],
    out_specs=pl.BlockSpec((1024, D), lambda i: (i, 0)),
    out_shape=jax.ShapeDtypeStruct((NP, D), jnp.bfloat16),
)


SCH = 16


@functools.partial(
    pl.kernel,
    out_type=jax.ShapeDtypeStruct((NC, NP, D), jnp.float32),
    mesh=_mesh,
    compiler_params=pltpu.CompilerParams(
        needs_layout_passes=False, use_tc_tiling_on_sc=False),
    scratch_types=[
        pltpu.VMEM((SCH, CH), jnp.int32),
        pltpu.VMEM((SCH, CH), jnp.int32),
        pltpu.VMEM((CH, DW), jnp.int32),
        pltpu.VMEM((CH, DW), jnp.int32),
        pltpu.VMEM((CH, D), jnp.float32),
        pltpu.VMEM_SHARED((NP, D), jnp.float32),
        pltpu.SemaphoreType.DMA,
        pltpu.SemaphoreType.DMA,
    ],
)
def _sc_aggregate(p_hbm, src_hbm, dst_hbm, out_hbm,
                  src_v, dst_v, g0, g1, gf, acc_sp, sg0, sg1):
    cid = lax.axis_index("c")
    sid = lax.axis_index("s")
    wid = sid * NC + cid

    def _zb(i, carry):
        gf[i // (D // 16), pl.ds((i % (D // 16)) * 16, 16)] = (
            jnp.zeros((16,), jnp.float32))
        return carry

    lax.fori_loop(0, CH * (D // 16), _zb, 0)
    for t in range(STRIPE // CH):
        pltpu.sync_copy(gf, acc_sp.at[pl.ds(sid * STRIPE + t * CH, CH)])
    plsc.subcore_barrier()

    def _convert(gbuf):
        def _rows(r0, carry):
            for i in range(16):
                r = r0 * 16 + i
                for k in range(DW // 16):
                    w = plsc.bitcast(gbuf[r, pl.ds(k * 16, 16)],
                                     jnp.bfloat16)
                    lo, hi = plsc.unpack(
                        w, format=plsc.PackFormat.INTERLEAVED)
                    gf[r, pl.ds(k * 16, 16)] = lo
                    gf[r, pl.ds(D // 2 + k * 16, 16)] = hi
            return carry

        lax.fori_loop(0, CH // 16, _rows, 0)

    def _super(sc_i, carry):
        pltpu.sync_copy(src_hbm.at[wid, pl.ds(sc_i * SCH, SCH)], src_v)
        pltpu.sync_copy(dst_hbm.at[wid, pl.ds(sc_i * SCH, SCH)], dst_v)
        pltpu.make_async_copy(p_hbm.at[src_v.at[0]], g0, sg0).start()
        pltpu.make_async_copy(p_hbm.at[src_v.at[1]], g1, sg1).start()

        def _body(jj, c2):
            j0 = jj * 2
            j1 = j0 + 1
            pltpu.make_async_copy(p_hbm.at[src_v.at[j0]], g0, sg0).wait()
            _convert(g0)

            @pl.when(jj < SCH // 2 - 1)
            def _():
                pltpu.make_async_copy(
                    p_hbm.at[src_v.at[j0 + 2]], g0, sg0).start()

            pltpu.sync_copy(gf, acc_sp.at[dst_v.at[j0]], add=True)
            pltpu.make_async_copy(p_hbm.at[src_v.at[j1]], g1, sg1).wait()
            _convert(g1)

            @pl.when(jj < SCH // 2 - 1)
            def _():
                pltpu.make_async_copy(
                    p_hbm.at[src_v.at[j1 + 2]], g1, sg1).start()

            pltpu.sync_copy(gf, acc_sp.at[dst_v.at[j1]], add=True)
            return c2

        lax.fori_loop(0, SCH // 2, _body, 0)
        return carry

    lax.fori_loop(0, EC // SCH, _super, 0)

    plsc.subcore_barrier()
    pltpu.sync_copy(acc_sp.at[pl.ds(sid * STRIPE, STRIPE)],
                    out_hbm.at[cid, pl.ds(sid * STRIPE, STRIPE)])


def _fin_body(acc_ref, p_ref, cnt_ref, o_ref):
    deg = cnt_ref[0, :] + cnt_ref[1, :] + 1.0
    norm = lax.rsqrt(deg)
    w = p_ref[...]
    lo = lax.bitcast_convert_type(w << 16, jnp.float32)
    hi = lax.bitcast_convert_type(
        w & jnp.int32(-65536), jnp.float32)
    p_perm = jnp.concatenate([lo, hi], axis=1)
    t = (acc_ref[0] + acc_ref[1] + p_perm) * norm[:, None]
    o_ref[...] = jnp.maximum(t, 0.0).astype(jnp.bfloat16)


_tc_finalize = pl.pallas_call(
    _fin_body,
    grid=(NP // 1024,),
    in_specs=[
        pl.BlockSpec((NC, 1024, D), lambda i: (0, i, 0)),
        pl.BlockSpec((1024, DW), lambda i: (i, 0)),
        pl.BlockSpec((NC, 1024), lambda i: (0, i)),
    ],
    out_specs=pl.BlockSpec((1024, D), lambda i: (i, 0)),
    out_shape=jax.ShapeDtypeStruct((NP, D), jnp.bfloat16),
)


@functools.partial(
    pl.kernel,
    out_type=jax.ShapeDtypeStruct((NW, LC, CH), jnp.float32),
    mesh=_mesh,
    compiler_params=pltpu.CompilerParams(
        needs_layout_passes=False, use_tc_tiling_on_sc=False),
    scratch_types=[
        pltpu.VMEM((LC, CH), jnp.int32),
        pltpu.VMEM((LC, CH), jnp.int32),
        pltpu.VMEM((CH, DW), jnp.int32),
        pltpu.VMEM((CH, DW), jnp.int32),
        pltpu.VMEM((CH, DW), jnp.int32),
        pltpu.VMEM((CH, DW), jnp.int32),
        pltpu.VMEM((LC, CH), jnp.float32),
        pltpu.VMEM_SHARED((NP, DW), jnp.int32),
        pltpu.SemaphoreType.DMA,
        pltpu.SemaphoreType.DMA,
    ],
)
def _sc_decode(z_hbm, s_hbm, d_hbm, out_hbm,
               s_v, d_v, a0, b0, a1, b1, o_v, z_sp, sm0, sm1):
    cid = lax.axis_index("c")
    sid = lax.axis_index("s")
    wid = sid * NC + cid
    pltpu.sync_copy(z_hbm.at[pl.ds(sid * STRIPE, STRIPE)],
                    z_sp.at[pl.ds(sid * STRIPE, STRIPE)])
    pltpu.sync_copy(s_hbm.at[wid], s_v)
    pltpu.sync_copy(d_hbm.at[wid], d_v)
    plsc.subcore_barrier()

    def _start(j, abuf, bbuf, sem):
        pltpu.make_async_copy(z_sp.at[s_v.at[j]], abuf, sem).start()
        pltpu.make_async_copy(z_sp.at[d_v.at[j]], bbuf, sem).start()

    def _wait(j, abuf, bbuf, sem):
        pltpu.make_async_copy(z_sp.at[s_v.at[j]], abuf, sem).wait()
        pltpu.make_async_copy(z_sp.at[d_v.at[j]], bbuf, sem).wait()

    lane = lax.iota(jnp.int32, 16)

    def _dotpair(abuf, bbuf, r, k):
        wa = plsc.bitcast(abuf[r, pl.ds(k * 16, 16)], jnp.bfloat16)
        wb = plsc.bitcast(bbuf[r, pl.ds(k * 16, 16)], jnp.bfloat16)
        a_lo, a_hi = plsc.unpack(wa, format=plsc.PackFormat.INTERLEAVED)
        b_lo, b_hi = plsc.unpack(wb, format=plsc.PackFormat.INTERLEAVED)
        return a_lo * b_lo + a_hi * b_hi

    def _compute(j, abuf, bbuf):
        def _grp(g, carry):
            res = jnp.zeros((16,), jnp.float32)
            for i in range(16):
                r = g * 16 + i
                acc = _dotpair(abuf, bbuf, r, 0)
                for k in range(1, DW // 16):
                    acc = acc + _dotpair(abuf, bbuf, r, k)
                res = res + jnp.where(lane == i, jnp.sum(acc), 0.0)
            o_v[j, pl.ds(g * 16, 16)] = res
            return carry

        lax.fori_loop(0, CH // 16, _grp, 0)

    _start(0, a0, b0, sm0)

    def _body(jj, carry):
        j0 = jj * 2
        j1 = j0 + 1
        _wait(j0, a0, b0, sm0)
        _start(j1, a1, b1, sm1)
        _compute(j0, a0, b0)
        _wait(j1, a1, b1, sm1)

        @pl.when(jj < LC // 2 - 1)
        def _():
            _start(j0 + 2, a0, b0, sm0)

        _compute(j1, a1, b1)
        return carry

    lax.fori_loop(0, LC // 2, _body, 0)
    pltpu.sync_copy(o_v, out_hbm.at[wid])


def kernel(x, edge_index, edge_label_index, W, b):
    ne = edge_index.shape[1]
    nl = edge_label_index.shape[1]
    ep = NW * EC * CH
    lp = NW * LC * CH

    src = jnp.concatenate(
        [edge_index[0], jnp.zeros((ep - ne,), jnp.int32)]).reshape(NW, EC, CH)
    dst = jnp.concatenate(
        [edge_index[1], jnp.full((ep - ne,), DUMMY, jnp.int32)]
    ).reshape(NW, EC, CH)
    s_idx = jnp.concatenate(
        [edge_label_index[0], jnp.zeros((lp - nl,), jnp.int32)]
    ).reshape(NW, LC, CH)
    d_idx = jnp.concatenate(
        [edge_label_index[1], jnp.zeros((lp - nl,), jnp.int32)]
    ).reshape(NW, LC, CH)
    x_pad = jnp.concatenate([x, jnp.zeros((NP - x.shape[0], D), x.dtype)])

    cnt = _sc_degree(dst)
    p = _tc_encode(x_pad, W, b.reshape(1, D), cnt)
    p32 = jax.lax.bitcast_convert_type(p.reshape(NP, DW, 2), jnp.int32)
    acc = _sc_aggregate(p32, src, dst)
    z = _tc_finalize(acc, p32, cnt)
    z32 = jax.lax.bitcast_convert_type(z.reshape(NP, DW, 2), jnp.int32)
    logits = _sc_decode(z32, s_idx, d_idx)
    return logits.reshape(-1)[:nl]


# speedup vs baseline: 2.8633x; 1.0401x over previous
import functools

import jax
import jax.numpy as jnp
from jax import lax
from jax.experimental import pallas as pl
from jax.experimental.pallas import tpu as pltpu
from jax.experimental.pallas import tpu_sc as plsc

N_NODES = 10000
NP = 10240
D = 128
NC = 2
NS = 16
NW = NC * NS
CH = 128
EC = 80
LC = 50
DW = D // 2
STRIPE = NP // NS
DUMMY = N_NODES

_mesh = plsc.VectorSubcoreMesh(core_axis_name="c", subcore_axis_name="s")


@functools.partial(
    pl.kernel,
    out_type=jax.ShapeDtypeStruct((NC, NP), jnp.float32),
    mesh=_mesh,
    scratch_types=[
        pltpu.VMEM((EC, CH), jnp.int32),
        pltpu.VMEM((CH,), jnp.float32),
        pltpu.VMEM((STRIPE,), jnp.float32),
        pltpu.VMEM_SHARED((NP,), jnp.float32),
    ],
)
def _sc_degree(dst_hbm, out_hbm, idx_v, ones_v, zeros_v, deg_sp):
    cid = lax.axis_index("c")
    sid = lax.axis_index("s")
    wid = sid * NC + cid

    for k in range(CH // 16):
        ones_v[pl.ds(k * 16, 16)] = jnp.full((16,), 1.0, jnp.float32)

    def _zb(i, carry):
        zeros_v[pl.ds(i * 16, 16)] = jnp.zeros((16,), jnp.float32)
        return carry

    lax.fori_loop(0, STRIPE // 16, _zb, 0)
    pltpu.sync_copy(zeros_v, deg_sp.at[pl.ds(sid * STRIPE, STRIPE)])
    plsc.subcore_barrier()

    pltpu.sync_copy(dst_hbm.at[wid], idx_v)

    def _body(j, carry):
        pltpu.sync_copy(ones_v, deg_sp.at[idx_v.at[j]], add=True)
        return carry

    lax.fori_loop(0, EC, _body, 0)
    plsc.subcore_barrier()
    pltpu.sync_copy(deg_sp.at[pl.ds(sid * STRIPE, STRIPE)],
                    out_hbm.at[cid, pl.ds(sid * STRIPE, STRIPE)])


def _enc_body(x_ref, w_ref, b_ref, cnt_ref, o_ref):
    h = jnp.dot(x_ref[...], w_ref[...], preferred_element_type=jnp.float32)
    h = h + b_ref[...]
    deg = cnt_ref[0, :] + cnt_ref[1, :] + 1.0
    norm = lax.rsqrt(deg)
    o_ref[...] = (h * norm[:, None]).astype(jnp.bfloat16)


_tc_encode = pl.pallas_call(
    _enc_body,
    grid=(NP // 1024,),
    in_specs=[
        pl.BlockSpec((1024, D), lambda i: (i, 0)),
        pl.BlockSpec((D, D), lambda i: (0, 0)),
        pl.BlockSpec((1, D), lambda i: (0, 0)),
        pl.BlockSpec((NC, 1024), lambda i: (0, i)),
    ],
    out_specs=pl.BlockSpec((1024, D), lambda i: (i, 0)),
    out_shape=jax.ShapeDtypeStruct((NP, D), jnp.bfloat16),
)


CA = 64
ECA = 160
SCH = 32


@functools.partial(
    pl.kernel,
    out_type=jax.ShapeDtypeStruct((NC, NP, D), jnp.float32),
    mesh=_mesh,
    compiler_params=pltpu.CompilerParams(
        needs_layout_passes=False, use_tc_tiling_on_sc=False),
    scratch_types=[
        pltpu.VMEM((SCH, CA), jnp.int32),
        pltpu.VMEM((SCH, CA), jnp.int32),
        pltpu.VMEM((CA, DW), jnp.int32),
        pltpu.VMEM((CA, DW), jnp.int32),
        pltpu.VMEM((CA, D), jnp.float32),
        pltpu.VMEM((CA, D), jnp.float32),
        pltpu.VMEM_SHARED((NP, D), jnp.float32),
        pltpu.SemaphoreType.DMA,
        pltpu.SemaphoreType.DMA,
        pltpu.SemaphoreType.DMA,
        pltpu.SemaphoreType.DMA,
    ],
)
def _sc_aggregate(p_hbm, src_hbm, dst_hbm, out_hbm,
                  src_v, dst_v, g0, g1, gf0, gf1, acc_sp,
                  sg0, sg1, ss0, ss1):
    cid = lax.axis_index("c")
    sid = lax.axis_index("s")
    wid = sid * NC + cid

    def _zb(i, carry):
        gf0[i // (D // 16), pl.ds((i % (D // 16)) * 16, 16)] = (
            jnp.zeros((16,), jnp.float32))
        return carry

    lax.fori_loop(0, CA * (D // 16), _zb, 0)
    for t in range(STRIPE // CA):
        pltpu.sync_copy(gf0, acc_sp.at[pl.ds(sid * STRIPE + t * CA, CA)])
    plsc.subcore_barrier()

    def _convert(gbuf, fbuf):
        def _rows(r0, carry):
            for i in range(16):
                r = r0 * 16 + i
                for k in range(DW // 16):
                    w = plsc.bitcast(gbuf[r, pl.ds(k * 16, 16)],
                                     jnp.bfloat16)
                    lo, hi = plsc.unpack(
                        w, format=plsc.PackFormat.INTERLEAVED)
                    fbuf[r, pl.ds(k * 16, 16)] = lo
                    fbuf[r, pl.ds(D // 2 + k * 16, 16)] = hi
            return carry

        lax.fori_loop(0, CA // 16, _rows, 0)

    def _half(j, gi, fb, sg, ss, jj):
        pltpu.make_async_copy(p_hbm.at[src_v.at[j]], gi, sg).wait()

        @pl.when(jj > 0)
        def _():
            pltpu.make_async_copy(
                fb, acc_sp.at[dst_v.at[j - 2]], ss).wait()

        _convert(gi, fb)

        @pl.when(j + 2 < SCH)
        def _():
            pltpu.make_async_copy(p_hbm.at[src_v.at[j + 2]], gi, sg).start()

        pltpu.async_copy(fb, acc_sp.at[dst_v.at[j]], ss, add=True)

    def _super(sc_i, carry):
        pltpu.sync_copy(src_hbm.at[wid, pl.ds(sc_i * SCH, SCH)], src_v)
        pltpu.sync_copy(dst_hbm.at[wid, pl.ds(sc_i * SCH, SCH)], dst_v)
        pltpu.make_async_copy(p_hbm.at[src_v.at[0]], g0, sg0).start()
        pltpu.make_async_copy(p_hbm.at[src_v.at[1]], g1, sg1).start()

        def _body(jj, c2):
            _half(jj * 2, g0, gf0, sg0, ss0, jj)
            _half(jj * 2 + 1, g1, gf1, sg1, ss1, jj)
            return c2

        lax.fori_loop(0, SCH // 2, _body, 0)
        pltpu.make_async_copy(
            gf0, acc_sp.at[dst_v.at[SCH - 2]], ss0).wait()
        pltpu.make_async_copy(
            gf1, acc_sp.at[dst_v.at[SCH - 1]], ss1).wait()
        return carry

    lax.fori_loop(0, ECA // SCH, _super, 0)

    plsc.subcore_barrier()
    pltpu.sync_copy(acc_sp.at[pl.ds(sid * STRIPE, STRIPE)],
                    out_hbm.at[cid, pl.ds(sid * STRIPE, STRIPE)])


def _fin_body(acc_ref, p_ref, cnt_ref, o_ref):
    deg = cnt_ref[0, :] + cnt_ref[1, :] + 1.0
    norm = lax.rsqrt(deg)
    w = p_ref[...]
    lo = lax.bitcast_convert_type(w << 16, jnp.float32)
    hi = lax.bitcast_convert_type(
        w & jnp.int32(-65536), jnp.float32)
    p_perm = jnp.concatenate([lo, hi], axis=1)
    t = (acc_ref[0] + acc_ref[1] + p_perm) * norm[:, None]
    o_ref[...] = jnp.maximum(t, 0.0).astype(jnp.bfloat16)


_tc_finalize = pl.pallas_call(
    _fin_body,
    grid=(NP // 1024,),
    in_specs=[
        pl.BlockSpec((NC, 1024, D), lambda i: (0, i, 0)),
        pl.BlockSpec((1024, DW), lambda i: (i, 0)),
        pl.BlockSpec((NC, 1024), lambda i: (0, i)),
    ],
    out_specs=pl.BlockSpec((1024, D), lambda i: (i, 0)),
    out_shape=jax.ShapeDtypeStruct((NP, D), jnp.bfloat16),
)


@functools.partial(
    pl.kernel,
    out_type=jax.ShapeDtypeStruct((NW, LC, CH), jnp.float32),
    mesh=_mesh,
    compiler_params=pltpu.CompilerParams(
        needs_layout_passes=False, use_tc_tiling_on_sc=False),
    scratch_types=[
        pltpu.VMEM((LC, CH), jnp.int32),
        pltpu.VMEM((LC, CH), jnp.int32),
        pltpu.VMEM((CH, DW), jnp.int32),
        pltpu.VMEM((CH, DW), jnp.int32),
        pltpu.VMEM((CH, DW), jnp.int32),
        pltpu.VMEM((CH, DW), jnp.int32),
        pltpu.VMEM((LC, CH), jnp.float32),
        pltpu.VMEM_SHARED((NP, DW), jnp.int32),
        pltpu.SemaphoreType.DMA,
        pltpu.SemaphoreType.DMA,
    ],
)
def _sc_decode(z_hbm, s_hbm, d_hbm, out_hbm,
               s_v, d_v, a0, b0, a1, b1, o_v, z_sp, sm0, sm1):
    cid = lax.axis_index("c")
    sid = lax.axis_index("s")
    wid = sid * NC + cid
    pltpu.sync_copy(z_hbm.at[pl.ds(sid * STRIPE, STRIPE)],
                    z_sp.at[pl.ds(sid * STRIPE, STRIPE)])
    pltpu.sync_copy(s_hbm.at[wid], s_v)
    pltpu.sync_copy(d_hbm.at[wid], d_v)
    plsc.subcore_barrier()

    def _start(j, abuf, bbuf, sem):
        pltpu.make_async_copy(z_sp.at[s_v.at[j]], abuf, sem).start()
        pltpu.make_async_copy(z_sp.at[d_v.at[j]], bbuf, sem).start()

    def _wait(j, abuf, bbuf, sem):
        pltpu.make_async_copy(z_sp.at[s_v.at[j]], abuf, sem).wait()
        pltpu.make_async_copy(z_sp.at[d_v.at[j]], bbuf, sem).wait()

    lane = lax.iota(jnp.int32, 16)

    def _dotpair(abuf, bbuf, r, k):
        wa = plsc.bitcast(abuf[r, pl.ds(k * 16, 16)], jnp.bfloat16)
        wb = plsc.bitcast(bbuf[r, pl.ds(k * 16, 16)], jnp.bfloat16)
        a_lo, a_hi = plsc.unpack(wa, format=plsc.PackFormat.INTERLEAVED)
        b_lo, b_hi = plsc.unpack(wb, format=plsc.PackFormat.INTERLEAVED)
        return a_lo * b_lo + a_hi * b_hi

    def _compute(j, abuf, bbuf):
        def _grp(g, carry):
            res = jnp.zeros((16,), jnp.float32)
            for i in range(16):
                r = g * 16 + i
                acc = _dotpair(abuf, bbuf, r, 0)
                for k in range(1, DW // 16):
                    acc = acc + _dotpair(abuf, bbuf, r, k)
                res = res + jnp.where(lane == i, jnp.sum(acc), 0.0)
            o_v[j, pl.ds(g * 16, 16)] = res
            return carry

        lax.fori_loop(0, CH // 16, _grp, 0)

    _start(0, a0, b0, sm0)

    def _body(jj, carry):
        j0 = jj * 2
        j1 = j0 + 1
        _wait(j0, a0, b0, sm0)
        _start(j1, a1, b1, sm1)
        _compute(j0, a0, b0)
        _wait(j1, a1, b1, sm1)

        @pl.when(jj < LC // 2 - 1)
        def _():
            _start(j0 + 2, a0, b0, sm0)

        _compute(j1, a1, b1)
        return carry

    lax.fori_loop(0, LC // 2, _body, 0)
    pltpu.sync_copy(o_v, out_hbm.at[wid])


def kernel(x, edge_index, edge_label_index, W, b):
    ne = edge_index.shape[1]
    nl = edge_label_index.shape[1]
    ep = NW * EC * CH
    lp = NW * LC * CH

    src = jnp.concatenate(
        [edge_index[0], jnp.zeros((ep - ne,), jnp.int32)]).reshape(NW, EC, CH)
    dst = jnp.concatenate(
        [edge_index[1], jnp.full((ep - ne,), DUMMY, jnp.int32)]
    ).reshape(NW, EC, CH)
    s_idx = jnp.concatenate(
        [edge_label_index[0], jnp.zeros((lp - nl,), jnp.int32)]
    ).reshape(NW, LC, CH)
    d_idx = jnp.concatenate(
        [edge_label_index[1], jnp.zeros((lp - nl,), jnp.int32)]
    ).reshape(NW, LC, CH)
    x_pad = jnp.concatenate([x, jnp.zeros((NP - x.shape[0], D), x.dtype)])

    cnt = _sc_degree(dst)
    p = _tc_encode(x_pad, W, b.reshape(1, D), cnt)
    p32 = jax.lax.bitcast_convert_type(p.reshape(NP, DW, 2), jnp.int32)
    acc = _sc_aggregate(p32, src.reshape(NW, ECA, CA),
                        dst.reshape(NW, ECA, CA))
    z = _tc_finalize(acc, p32, cnt)
    z32 = jax.lax.bitcast_convert_type(z.reshape(NP, DW, 2), jnp.int32)
    logits = _sc_decode(z32, s_idx, d_idx)
    return logits.reshape(-1)[:nl]


# speedup vs baseline: 2.9262x; 1.0220x over previous
import functools

import jax
import jax.numpy as jnp
from jax import lax
from jax.experimental import pallas as pl
from jax.experimental.pallas import tpu as pltpu
from jax.experimental.pallas import tpu_sc as plsc

N_NODES = 10000
NP = 10240
D = 128
NC = 2
NS = 16
NW = NC * NS
CH = 128
EC = 80
LC = 50
DW = D // 2
STRIPE = NP // NS
DUMMY = N_NODES

_mesh = plsc.VectorSubcoreMesh(core_axis_name="c", subcore_axis_name="s")


@functools.partial(
    pl.kernel,
    out_type=jax.ShapeDtypeStruct((NC, NP), jnp.float32),
    mesh=_mesh,
    scratch_types=[
        pltpu.VMEM((EC, CH), jnp.int32),
        pltpu.VMEM((CH,), jnp.float32),
        pltpu.VMEM((STRIPE,), jnp.float32),
        pltpu.VMEM_SHARED((NP,), jnp.float32),
        pltpu.SemaphoreType.DMA,
    ],
)
def _sc_degree(dst_hbm, out_hbm, idx_v, ones_v, zeros_v, deg_sp, sd):
    cid = lax.axis_index("c")
    sid = lax.axis_index("s")
    wid = sid * NC + cid

    for k in range(CH // 16):
        ones_v[pl.ds(k * 16, 16)] = jnp.full((16,), 1.0, jnp.float32)

    def _zb(i, carry):
        zeros_v[pl.ds(i * 16, 16)] = jnp.zeros((16,), jnp.float32)
        return carry

    lax.fori_loop(0, STRIPE // 16, _zb, 0)
    pltpu.sync_copy(zeros_v, deg_sp.at[pl.ds(sid * STRIPE, STRIPE)])
    plsc.subcore_barrier()

    pltpu.sync_copy(dst_hbm.at[wid], idx_v)

    def _body(j, carry):
        pltpu.async_copy(ones_v, deg_sp.at[idx_v.at[j]], sd, add=True)
        return carry

    lax.fori_loop(0, EC, _body, 0)

    def _drain(j, carry):
        pltpu.make_async_copy(ones_v, deg_sp.at[idx_v.at[j]], sd).wait()
        return carry

    lax.fori_loop(0, EC, _drain, 0)
    plsc.subcore_barrier()
    pltpu.sync_copy(deg_sp.at[pl.ds(sid * STRIPE, STRIPE)],
                    out_hbm.at[cid, pl.ds(sid * STRIPE, STRIPE)])


def _enc_body(x_ref, w_ref, b_ref, cnt_ref, o_ref):
    h = jnp.dot(x_ref[...], w_ref[...], preferred_element_type=jnp.float32)
    h = h + b_ref[...]
    deg = cnt_ref[0, :] + cnt_ref[1, :] + 1.0
    norm = lax.rsqrt(deg)
    o_ref[...] = (h * norm[:, None]).astype(jnp.bfloat16)


_tc_encode = pl.pallas_call(
    _enc_body,
    grid=(NP // 1024,),
    in_specs=[
        pl.BlockSpec((1024, D), lambda i: (i, 0)),
        pl.BlockSpec((D, D), lambda i: (0, 0)),
        pl.BlockSpec((1, D), lambda i: (0, 0)),
        pl.BlockSpec((NC, 1024), lambda i: (0, i)),
    ],
    out_specs=pl.BlockSpec((1024, D), lambda i: (i, 0)),
    out_shape=jax.ShapeDtypeStruct((NP, D), jnp.bfloat16),
)


CA = 64
ECA = 160
SCH = 80


@functools.partial(
    pl.kernel,
    out_type=jax.ShapeDtypeStruct((NC, NP, D), jnp.float32),
    mesh=_mesh,
    compiler_params=pltpu.CompilerParams(
        needs_layout_passes=False, use_tc_tiling_on_sc=False),
    scratch_types=[
        pltpu.VMEM((SCH, CA), jnp.int32),
        pltpu.VMEM((SCH, CA), jnp.int32),
        pltpu.VMEM((CA, DW), jnp.int32),
        pltpu.VMEM((CA, DW), jnp.int32),
        pltpu.VMEM((CA, D), jnp.float32),
        pltpu.VMEM((CA, D), jnp.float32),
        pltpu.VMEM_SHARED((NP, D), jnp.float32),
        pltpu.SemaphoreType.DMA,
        pltpu.SemaphoreType.DMA,
        pltpu.SemaphoreType.DMA,
        pltpu.SemaphoreType.DMA,
    ],
)
def _sc_aggregate(p_hbm, src_hbm, dst_hbm, out_hbm,
                  src_v, dst_v, g0, g1, gf0, gf1, acc_sp,
                  sg0, sg1, ss0, ss1):
    cid = lax.axis_index("c")
    sid = lax.axis_index("s")
    wid = sid * NC + cid

    def _zb(i, carry):
        gf0[i // (D // 16), pl.ds((i % (D // 16)) * 16, 16)] = (
            jnp.zeros((16,), jnp.float32))
        return carry

    lax.fori_loop(0, CA * (D // 16), _zb, 0)
    for t in range(STRIPE // CA):
        pltpu.sync_copy(gf0, acc_sp.at[pl.ds(sid * STRIPE + t * CA, CA)])
    plsc.subcore_barrier()

    def _convert(gbuf, fbuf):
        def _rows(r0, carry):
            for i in range(16):
                r = r0 * 16 + i
                for k in range(DW // 16):
                    w = plsc.bitcast(gbuf[r, pl.ds(k * 16, 16)],
                                     jnp.bfloat16)
                    lo, hi = plsc.unpack(
                        w, format=plsc.PackFormat.INTERLEAVED)
                    fbuf[r, pl.ds(k * 16, 16)] = lo
                    fbuf[r, pl.ds(D // 2 + k * 16, 16)] = hi
            return carry

        lax.fori_loop(0, CA // 16, _rows, 0)

    def _half(j, gi, fb, sg, ss, jj):
        pltpu.make_async_copy(p_hbm.at[src_v.at[j]], gi, sg).wait()

        @pl.when(jj > 0)
        def _():
            pltpu.make_async_copy(
                fb, acc_sp.at[dst_v.at[j - 2]], ss).wait()

        _convert(gi, fb)

        @pl.when(j + 2 < SCH)
        def _():
            pltpu.make_async_copy(p_hbm.at[src_v.at[j + 2]], gi, sg).start()

        pltpu.async_copy(fb, acc_sp.at[dst_v.at[j]], ss, add=True)

    def _super(sc_i, carry):
        pltpu.sync_copy(src_hbm.at[wid, pl.ds(sc_i * SCH, SCH)], src_v)
        pltpu.sync_copy(dst_hbm.at[wid, pl.ds(sc_i * SCH, SCH)], dst_v)
        pltpu.make_async_copy(p_hbm.at[src_v.at[0]], g0, sg0).start()
        pltpu.make_async_copy(p_hbm.at[src_v.at[1]], g1, sg1).start()

        def _body(jj, c2):
            _half(jj * 2, g0, gf0, sg0, ss0, jj)
            _half(jj * 2 + 1, g1, gf1, sg1, ss1, jj)
            return c2

        lax.fori_loop(0, SCH // 2, _body, 0)
        pltpu.make_async_copy(
            gf0, acc_sp.at[dst_v.at[SCH - 2]], ss0).wait()
        pltpu.make_async_copy(
            gf1, acc_sp.at[dst_v.at[SCH - 1]], ss1).wait()
        return carry

    lax.fori_loop(0, ECA // SCH, _super, 0)

    plsc.subcore_barrier()
    pltpu.sync_copy(acc_sp.at[pl.ds(sid * STRIPE, STRIPE)],
                    out_hbm.at[cid, pl.ds(sid * STRIPE, STRIPE)])


def _fin_body(acc_ref, p_ref, cnt_ref, o_ref):
    deg = cnt_ref[0, :] + cnt_ref[1, :] + 1.0
    norm = lax.rsqrt(deg)
    w = p_ref[...]
    lo = lax.bitcast_convert_type(w << 16, jnp.float32)
    hi = lax.bitcast_convert_type(
        w & jnp.int32(-65536), jnp.float32)
    p_perm = jnp.concatenate([lo, hi], axis=1)
    t = (acc_ref[0] + acc_ref[1] + p_perm) * norm[:, None]
    o_ref[...] = jnp.maximum(t, 0.0).astype(jnp.bfloat16)


_tc_finalize = pl.pallas_call(
    _fin_body,
    grid=(NP // 1024,),
    in_specs=[
        pl.BlockSpec((NC, 1024, D), lambda i: (0, i, 0)),
        pl.BlockSpec((1024, DW), lambda i: (i, 0)),
        pl.BlockSpec((NC, 1024), lambda i: (0, i)),
    ],
    out_specs=pl.BlockSpec((1024, D), lambda i: (i, 0)),
    out_shape=jax.ShapeDtypeStruct((NP, D), jnp.bfloat16),
)


@functools.partial(
    pl.kernel,
    out_type=jax.ShapeDtypeStruct((NW, LC, CH), jnp.float32),
    mesh=_mesh,
    compiler_params=pltpu.CompilerParams(
        needs_layout_passes=False, use_tc_tiling_on_sc=False),
    scratch_types=[
        pltpu.VMEM((LC, CH), jnp.int32),
        pltpu.VMEM((LC, CH), jnp.int32),
        pltpu.VMEM((CH, DW), jnp.int32),
        pltpu.VMEM((CH, DW), jnp.int32),
        pltpu.VMEM((CH, DW), jnp.int32),
        pltpu.VMEM((CH, DW), jnp.int32),
        pltpu.VMEM((LC, CH), jnp.float32),
        pltpu.VMEM_SHARED((NP, DW), jnp.int32),
        pltpu.SemaphoreType.DMA,
        pltpu.SemaphoreType.DMA,
    ],
)
def _sc_decode(z_hbm, s_hbm, d_hbm, out_hbm,
               s_v, d_v, a0, b0, a1, b1, o_v, z_sp, sm0, sm1):
    cid = lax.axis_index("c")
    sid = lax.axis_index("s")
    wid = sid * NC + cid
    pltpu.sync_copy(z_hbm.at[pl.ds(sid * STRIPE, STRIPE)],
                    z_sp.at[pl.ds(sid * STRIPE, STRIPE)])
    pltpu.sync_copy(s_hbm.at[wid], s_v)
    pltpu.sync_copy(d_hbm.at[wid], d_v)
    plsc.subcore_barrier()

    def _start(j, abuf, bbuf, sem):
        pltpu.make_async_copy(z_sp.at[s_v.at[j]], abuf, sem).start()
        pltpu.make_async_copy(z_sp.at[d_v.at[j]], bbuf, sem).start()

    def _wait(j, abuf, bbuf, sem):
        pltpu.make_async_copy(z_sp.at[s_v.at[j]], abuf, sem).wait()
        pltpu.make_async_copy(z_sp.at[d_v.at[j]], bbuf, sem).wait()

    lane = lax.iota(jnp.int32, 16)

    def _dotpair(abuf, bbuf, r, k):
        wa = plsc.bitcast(abuf[r, pl.ds(k * 16, 16)], jnp.bfloat16)
        wb = plsc.bitcast(bbuf[r, pl.ds(k * 16, 16)], jnp.bfloat16)
        a_lo, a_hi = plsc.unpack(wa, format=plsc.PackFormat.INTERLEAVED)
        b_lo, b_hi = plsc.unpack(wb, format=plsc.PackFormat.INTERLEAVED)
        return a_lo * b_lo + a_hi * b_hi

    def _compute(j, abuf, bbuf):
        def _grp(g, carry):
            res = jnp.zeros((16,), jnp.float32)
            for i in range(16):
                r = g * 16 + i
                acc = _dotpair(abuf, bbuf, r, 0)
                for k in range(1, DW // 16):
                    acc = acc + _dotpair(abuf, bbuf, r, k)
                res = res + jnp.where(lane == i, jnp.sum(acc), 0.0)
            o_v[j, pl.ds(g * 16, 16)] = res
            return carry

        lax.fori_loop(0, CH // 16, _grp, 0)

    _start(0, a0, b0, sm0)

    def _body(jj, carry):
        j0 = jj * 2
        j1 = j0 + 1
        _wait(j0, a0, b0, sm0)
        _start(j1, a1, b1, sm1)
        _compute(j0, a0, b0)
        _wait(j1, a1, b1, sm1)

        @pl.when(jj < LC // 2 - 1)
        def _():
            _start(j0 + 2, a0, b0, sm0)

        _compute(j1, a1, b1)
        return carry

    lax.fori_loop(0, LC // 2, _body, 0)
    pltpu.sync_copy(o_v, out_hbm.at[wid])


def kernel(x, edge_index, edge_label_index, W, b):
    ne = edge_index.shape[1]
    nl = edge_label_index.shape[1]
    ep = NW * EC * CH
    lp = NW * LC * CH

    src = jnp.concatenate(
        [edge_index[0], jnp.zeros((ep - ne,), jnp.int32)]).reshape(NW, EC, CH)
    dst = jnp.concatenate(
        [edge_index[1], jnp.full((ep - ne,), DUMMY, jnp.int32)]
    ).reshape(NW, EC, CH)
    s_idx = jnp.concatenate(
        [edge_label_index[0], jnp.zeros((lp - nl,), jnp.int32)]
    ).reshape(NW, LC, CH)
    d_idx = jnp.concatenate(
        [edge_label_index[1], jnp.zeros((lp - nl,), jnp.int32)]
    ).reshape(NW, LC, CH)
    x_pad = jnp.concatenate([x, jnp.zeros((NP - x.shape[0], D), x.dtype)])

    cnt = _sc_degree(dst)
    p = _tc_encode(x_pad, W, b.reshape(1, D), cnt)
    p32 = jax.lax.bitcast_convert_type(p.reshape(NP, DW, 2), jnp.int32)
    acc = _sc_aggregate(p32, src.reshape(NW, ECA, CA),
                        dst.reshape(NW, ECA, CA))
    z = _tc_finalize(acc, p32, cnt)
    z32 = jax.lax.bitcast_convert_type(z.reshape(NP, DW, 2), jnp.int32)
    logits = _sc_decode(z32, s_idx, d_idx)
    return logits.reshape(-1)[:nl]
